# Initial kernel scaffold; baseline (speedup 1.0000x reference)
import functools

import jax
import jax.numpy as jnp
from jax import lax
from jax.experimental import pallas as pl
from jax.experimental.pallas import tpu as pltpu
from jax.experimental.pallas import tpu_sc as plsc

NC = 2
NS = 16
L = 16
NW = NC * NS


def _head_body(x_ref, w_ref, asr_ref, adr_ref, h_ref, as_ref, ad_ref,
               m_ref, ws_ref):
    h = jnp.dot(x_ref[...], w_ref[...], preferred_element_type=jnp.float32)
    h_ref[...] = h
    as_ = jnp.sum(h * asr_ref[...], axis=1, keepdims=True)
    ad_ = jnp.sum(h * adr_ref[...], axis=1, keepdims=True)
    as_ref[...] = as_
    ad_ref[...] = ad_
    m = jnp.max(as_) + jnp.max(ad_)
    m = jnp.where(m >= 0.0, m, 0.2 * m)
    m_ref[...] = jnp.full((1, L), m, jnp.float32)
    y = as_ + ad_
    z = jnp.maximum(y, 0.2 * y)
    ws_ref[...] = jnp.exp(z - m)


def _head(x, W, a_src, a_dst):
    N, D = x.shape
    f32 = jnp.float32
    return pl.pallas_call(
        _head_body,
        out_shape=[
            jax.ShapeDtypeStruct((N, D), f32),
            jax.ShapeDtypeStruct((N, 1), f32),
            jax.ShapeDtypeStruct((N, 1), f32),
            jax.ShapeDtypeStruct((1, L), f32),
            jax.ShapeDtypeStruct((N, 1), f32),
        ],
    )(x, W, a_src.reshape(1, D), a_dst.reshape(1, D))


def _combine_body(acc_ref, h_ref, ws_ref, b_ref, o_ref, *, act, D):
    ws = ws_ref[...]
    acc2 = acc_ref[0] + acc_ref[1]
    num = acc2[:, :D] + ws * h_ref[...]
    den = acc2[:, D:D + 1] + ws
    out = num / (den + 1e-16) + b_ref[...]
    if act == "relu":
        out = jnp.maximum(out, 0.0)
    else:
        mx = jnp.max(out, axis=1, keepdims=True)
        sh = out - mx
        out = sh - jnp.log(jnp.sum(jnp.exp(sh), axis=1, keepdims=True))
    o_ref[...] = out


def _combine(acc, h, ws, b, act):
    N, D = h.shape
    return pl.pallas_call(
        functools.partial(_combine_body, act=act, D=D),
        out_shape=jax.ShapeDtypeStruct((N, D), jnp.float32),
    )(acc, h, ws, b.reshape(1, D))


def _sc_body(src_hbm, dst_hbm, as_hbm, ad_hbm, m_hbm, h_hbm, z_hbm,
             acc_out, as_v, ad_v, m_v, sidx_v, didx_v, w_v, rows_v,
             rowsw_v, acc_sh, sem, *, N, D, E, C):
    RPT = N // NS
    EPW = E // NW
    NCHUNK = EPW // C

    c = lax.axis_index("c")
    s = lax.axis_index("s")
    wid = s * NC + c

    pltpu.sync_copy(as_hbm, as_v)
    pltpu.sync_copy(ad_hbm, ad_v)
    pltpu.sync_copy(m_hbm, m_v)
    pltpu.sync_copy(z_hbm, acc_sh.at[pl.ds(s * RPT, RPT)])
    plsc.subcore_barrier()

    iota = lax.iota(jnp.int32, L)
    mask0 = iota == 0
    zero16 = jnp.zeros((L,), jnp.int32)
    mvec = m_v[0, :]
    base = wid * EPW

    def chunk(g, carry):
        off = base + g * C
        pltpu.sync_copy(src_hbm.at[pl.ds(off, C)], sidx_v)
        pltpu.sync_copy(dst_hbm.at[pl.ds(off, C)], didx_v)
        cp = pltpu.async_copy(h_hbm.at[sidx_v], rows_v, sem)
        for j in range(C // L):
            si = sidx_v[pl.ds(j * L, L)]
            di = didx_v[pl.ds(j * L, L)]
            a = plsc.load_gather(as_v, [si, zero16])
            d = plsc.load_gather(ad_v, [di, zero16])
            y = a + d
            z = jnp.maximum(y, 0.2 * y)
            w_v[pl.ds(j * L, L)] = jnp.exp(z - mvec)
        cp.wait()

        def srow(j, carry2):
            wj = plsc.load_gather(w_v, [jnp.full((L,), j, jnp.int32)])
            for k in range(D // L):
                rowsw_v[j, pl.ds(k * L, L)] = rows_v[j, pl.ds(k * L, L)] * wj
            rowsw_v[j, pl.ds(D, L)] = jnp.where(mask0, wj, 0.0)
            return carry2

        lax.fori_loop(0, C, srow, 0)
        pltpu.sync_copy(rowsw_v, acc_sh.at[didx_v], add=True)
        return carry

    lax.fori_loop(0, NCHUNK, chunk, 0)
    plsc.subcore_barrier()
    pltpu.sync_copy(acc_sh.at[pl.ds(s * RPT, RPT)],
                    acc_out.at[c, pl.ds(s * RPT, RPT)])


def _sc_edges(src, dst, as_, ad_, m, h, zeros_nd):
    N, D = h.shape
    E = src.shape[0]
    C = 80
    assert E % (NW * C) == 0 and N % NS == 0
    DW = D + L
    f32 = jnp.float32
    mesh = plsc.VectorSubcoreMesh(core_axis_name="c", subcore_axis_name="s",
                                  num_cores=NC, num_subcores=NS)
    body = functools.partial(_sc_body, N=N, D=D, E=E, C=C)
    fn = pl.kernel(
        body,
        out_type=jax.ShapeDtypeStruct((NC, N, DW), f32),
        mesh=mesh,
        scratch_types=[
            pltpu.VMEM((N, 1), f32),
            pltpu.VMEM((N, 1), f32),
            pltpu.VMEM((1, L), f32),
            pltpu.VMEM((C,), jnp.int32),
            pltpu.VMEM((C,), jnp.int32),
            pltpu.VMEM((C,), f32),
            pltpu.VMEM((C, D), f32),
            pltpu.VMEM((C, DW), f32),
            pltpu.VMEM_SHARED((N, DW), f32),
            pltpu.SemaphoreType.DMA,
        ],
    )
    return fn(src, dst, as_, ad_, m, h, zeros_nd)


def _gat_layer(x, edge_src, edge_dst, W, a_src, a_dst, b, zeros_nd, act):
    h, as_, ad_, m, ws = _head(x, W, a_src, a_dst)
    acc = _sc_edges(edge_src, edge_dst, as_, ad_, m, h, zeros_nd)
    return _combine(acc, h, ws, b, act)


def kernel(x, edge_index, W1, a_src1, a_dst1, b1, W2, a_src2, a_dst2, b2):
    N, D = x.shape
    src = edge_index[0]
    dst = edge_index[1]
    zeros_nd = jnp.zeros((N // NS, D + L), jnp.float32)
    h1 = _gat_layer(x, src, dst, W1, a_src1, a_dst1, b1, zeros_nd, "relu")
    return _gat_layer(h1, src, dst, W2, a_src2, a_dst2, b2, zeros_nd,
                      "logsoftmax")


# speedup vs baseline: 22.9309x; 22.9309x over previous
import functools

import jax
import jax.numpy as jnp
from jax import lax
from jax.experimental import pallas as pl
from jax.experimental.pallas import tpu as pltpu
from jax.experimental.pallas import tpu_sc as plsc

NC = 2
NS = 16
L = 16
NW = NC * NS


def _head_body(x_ref, w_ref, asr_ref, adr_ref, h_ref, as_ref, ad_ref,
               m_ref, ws_ref):
    h = jnp.dot(x_ref[...], w_ref[...], preferred_element_type=jnp.float32)
    h_ref[...] = h
    as_ = jnp.sum(h * asr_ref[...], axis=1, keepdims=True)
    ad_ = jnp.sum(h * adr_ref[...], axis=1, keepdims=True)
    as_ref[...] = as_
    ad_ref[...] = ad_
    m = jnp.max(as_) + jnp.max(ad_)
    m = jnp.where(m >= 0.0, m, 0.2 * m)
    m_ref[...] = jnp.full((1, L), m, jnp.float32)
    y = as_ + ad_
    z = jnp.maximum(y, 0.2 * y)
    ws_ref[...] = jnp.exp(z - m)


def _head(x, W, a_src, a_dst):
    N, D = x.shape
    f32 = jnp.float32
    return pl.pallas_call(
        _head_body,
        out_shape=[
            jax.ShapeDtypeStruct((N, D), f32),
            jax.ShapeDtypeStruct((N, 1), f32),
            jax.ShapeDtypeStruct((N, 1), f32),
            jax.ShapeDtypeStruct((1, L), f32),
            jax.ShapeDtypeStruct((N, 1), f32),
        ],
    )(x, W, a_src.reshape(1, D), a_dst.reshape(1, D))


def _combine_body(acc_ref, dent_ref, h_ref, ws_ref, b_ref, o_ref, *, act, D):
    ws = ws_ref[...]
    N = ws_ref.shape[0]
    num = acc_ref[0, :N, :] + acc_ref[1, :N, :] + ws * h_ref[...]
    den = jnp.sum(dent_ref[...], axis=1, keepdims=True) + ws
    out = num / (den + 1e-16) + b_ref[...]
    if act == "relu":
        out = jnp.maximum(out, 0.0)
    else:
        mx = jnp.max(out, axis=1, keepdims=True)
        sh = out - mx
        out = sh - jnp.log(jnp.sum(jnp.exp(sh), axis=1, keepdims=True))
    o_ref[...] = out


def _combine(acc, den_parts, h, ws, b, act):
    N, D = h.shape
    den_t = den_parts.T
    return pl.pallas_call(
        functools.partial(_combine_body, act=act, D=D),
        out_shape=jax.ShapeDtypeStruct((N, D), jnp.float32),
    )(acc, den_t, h, ws, b.reshape(1, D))


def _sc_body(src_hbm, dst_hbm, as_hbm, ad_hbm, m_hbm, h_hbm, z_hbm, zn_hbm,
             acc_out, den_out, as_v, ad_v, m_v, den_v, sidx_v, didx_v, w_v,
             rows_v, acc_sh, sem, *, N, NP, D, E, C):
    RPT = NP // NS
    EPW = E // NW
    NCHUNK = EPW // C

    c = lax.axis_index("c")
    s = lax.axis_index("s")
    wid = s * NC + c

    pltpu.sync_copy(as_hbm, as_v)
    pltpu.sync_copy(ad_hbm, ad_v)
    pltpu.sync_copy(m_hbm, m_v)
    pltpu.sync_copy(zn_hbm, den_v)
    pltpu.sync_copy(z_hbm, acc_sh.at[pl.ds(s * RPT, RPT)])
    plsc.subcore_barrier()

    mvec = m_v[...]
    base = wid * EPW

    def chunk(g, carry):
        off = base + g * C
        pltpu.sync_copy(src_hbm.at[pl.ds(off, C)], sidx_v)
        pltpu.sync_copy(dst_hbm.at[pl.ds(off, C)], didx_v)
        cp = pltpu.async_copy(h_hbm.at[sidx_v], rows_v, sem)
        for j in range(C // L):
            si = sidx_v[pl.ds(j * L, L)]
            di = didx_v[pl.ds(j * L, L)]
            a = plsc.load_gather(as_v, [si])
            d = plsc.load_gather(ad_v, [di])
            y = a + d
            z = jnp.maximum(y, 0.2 * y)
            w = jnp.exp(z - mvec)
            w_v[pl.ds(j * L, L)] = w
            plsc.addupdate_scatter(den_v, [di], w)
        cp.wait()

        def srow(j, carry2):
            wj = plsc.load_gather(w_v, [jnp.full((L,), j, jnp.int32)])
            for k in range(D // L):
                rows_v[j, pl.ds(k * L, L)] = rows_v[j, pl.ds(k * L, L)] * wj
            return carry2

        lax.fori_loop(0, C, srow, 0)
        pltpu.sync_copy(rows_v, acc_sh.at[didx_v], add=True)
        return carry

    lax.fori_loop(0, NCHUNK, chunk, 0)
    pltpu.sync_copy(den_v, den_out.at[wid])
    plsc.subcore_barrier()
    pltpu.sync_copy(acc_sh.at[pl.ds(s * RPT, RPT)],
                    acc_out.at[c, pl.ds(s * RPT, RPT)])


def _sc_edges(src, dst, as_, ad_, m, h, zeros_nd, zeros_n):
    N, D = h.shape
    E = src.shape[0]
    C = 80
    NP = ((N + NS * 8 - 1) // (NS * 8)) * NS * 8
    assert E % (NW * C) == 0
    f32 = jnp.float32
    mesh = plsc.VectorSubcoreMesh(core_axis_name="c", subcore_axis_name="s",
                                  num_cores=NC, num_subcores=NS)
    body = functools.partial(_sc_body, N=N, NP=NP, D=D, E=E, C=C)
    fn = pl.kernel(
        body,
        out_type=[
            jax.ShapeDtypeStruct((NC, NP, D), f32),
            jax.ShapeDtypeStruct((NW, N), f32),
        ],
        mesh=mesh,
        compiler_params=pltpu.CompilerParams(needs_layout_passes=False),
        scratch_types=[
            pltpu.VMEM((N,), f32),
            pltpu.VMEM((N,), f32),
            pltpu.VMEM((L,), f32),
            pltpu.VMEM((N,), f32),
            pltpu.VMEM((C,), jnp.int32),
            pltpu.VMEM((C,), jnp.int32),
            pltpu.VMEM((C,), f32),
            pltpu.VMEM((C, D), f32),
            pltpu.VMEM_SHARED((NP, D), f32),
            pltpu.SemaphoreType.DMA,
        ],
    )
    return fn(src, dst, as_.reshape(N), ad_.reshape(N), m.reshape(L), h,
              zeros_nd, zeros_n)


def _gat_layer(x, edge_src, edge_dst, W, a_src, a_dst, b, zeros_nd, zeros_n,
               act):
    h, as_, ad_, m, ws = _head(x, W, a_src, a_dst)
    acc, den_parts = _sc_edges(edge_src, edge_dst, as_, ad_, m, h, zeros_nd,
                               zeros_n)
    return _combine(acc, den_parts, h, ws, b, act)


def kernel(x, edge_index, W1, a_src1, a_dst1, b1, W2, a_src2, a_dst2, b2):
    N, D = x.shape
    src = edge_index[0]
    dst = edge_index[1]
    NP = ((N + NS * 8 - 1) // (NS * 8)) * NS * 8
    zeros_nd = jnp.zeros((NP // NS, D), jnp.float32)
    zeros_n = jnp.zeros((N,), jnp.float32)
    h1 = _gat_layer(x, src, dst, W1, a_src1, a_dst1, b1, zeros_nd, zeros_n,
                    "relu")
    return _gat_layer(h1, src, dst, W2, a_src2, a_dst2, b2, zeros_nd,
                      zeros_n, "logsoftmax")


# speedup vs baseline: 49.4259x; 2.1554x over previous
import functools

import jax
import jax.numpy as jnp
from jax import lax
from jax.experimental import pallas as pl
from jax.experimental.pallas import tpu as pltpu
from jax.experimental.pallas import tpu_sc as plsc

NC = 2
NS = 16
L = 16
NW = NC * NS


def _head_body(x_ref, w_ref, asr_ref, adr_ref, h_ref, as_ref, ad_ref,
               m_ref, ws_ref):
    h = jnp.dot(x_ref[...], w_ref[...], preferred_element_type=jnp.float32)
    h_ref[...] = h
    as_ = jnp.sum(h * asr_ref[...], axis=1, keepdims=True)
    ad_ = jnp.sum(h * adr_ref[...], axis=1, keepdims=True)
    as_ref[...] = as_
    ad_ref[...] = ad_
    m = jnp.max(as_) + jnp.max(ad_)
    m = jnp.where(m >= 0.0, m, 0.2 * m)
    m_ref[...] = jnp.full((1, L), m, jnp.float32)
    y = as_ + ad_
    z = jnp.maximum(y, 0.2 * y)
    ws_ref[...] = jnp.exp(z - m)


def _head(x, W, a_src, a_dst):
    N, D = x.shape
    f32 = jnp.float32
    return pl.pallas_call(
        _head_body,
        out_shape=[
            jax.ShapeDtypeStruct((N, D), f32),
            jax.ShapeDtypeStruct((N, 1), f32),
            jax.ShapeDtypeStruct((N, 1), f32),
            jax.ShapeDtypeStruct((1, L), f32),
            jax.ShapeDtypeStruct((N, 1), f32),
        ],
    )(x, W, a_src.reshape(1, D), a_dst.reshape(1, D))


def _combine_body(acc_ref, dent_ref, h_ref, ws_ref, b_ref, o_ref, *, act, D):
    ws = ws_ref[...]
    N = ws_ref.shape[0]
    num = acc_ref[0, :N, :] + acc_ref[1, :N, :] + ws * h_ref[...]
    den = jnp.sum(dent_ref[...], axis=1, keepdims=True) + ws
    out = num / (den + 1e-16) + b_ref[...]
    if act == "relu":
        out = jnp.maximum(out, 0.0)
    else:
        mx = jnp.max(out, axis=1, keepdims=True)
        sh = out - mx
        out = sh - jnp.log(jnp.sum(jnp.exp(sh), axis=1, keepdims=True))
    o_ref[...] = out


def _combine(acc, den_parts, h, ws, b, act):
    N, D = h.shape
    den_t = den_parts.T
    return pl.pallas_call(
        functools.partial(_combine_body, act=act, D=D),
        out_shape=jax.ShapeDtypeStruct((N, D), jnp.float32),
    )(acc, den_t, h, ws, b.reshape(1, D))


NBUF = 3


def _sc_body(src_hbm, dst_hbm, as_hbm, ad_hbm, m_hbm, h_hbm, z_hbm, zn_hbm,
             acc_out, den_out, m_v, den_v, *rings, N, NP, D, E, C):
    RPT = NP // NS
    EPW = E // NW
    NCHUNK = EPW // C
    sidx_r = rings[0:NBUF]
    didx_r = rings[NBUF:2 * NBUF]
    av_r = rings[2 * NBUF:3 * NBUF]
    dv_r = rings[3 * NBUF:4 * NBUF]
    w_r = rings[4 * NBUF:5 * NBUF]
    dscat_r = rings[5 * NBUF:6 * NBUF]
    grows = rings[6 * NBUF:7 * NBUF]
    acc_sh = rings[7 * NBUF]
    sem_i = rings[7 * NBUF + 1:8 * NBUF + 1]
    sem_g = rings[8 * NBUF + 1:9 * NBUF + 1]
    sem_s = rings[9 * NBUF + 1:10 * NBUF + 1]

    c = lax.axis_index("c")
    s = lax.axis_index("s")
    wid = s * NC + c
    base = wid * EPW

    pltpu.sync_copy(m_hbm, m_v)
    pltpu.sync_copy(zn_hbm, den_v)
    pltpu.sync_copy(z_hbm, acc_sh.at[pl.ds(s * RPT, RPT)])
    plsc.subcore_barrier()
    mvec = m_v[...]

    def issue_idx(g, b):
        pltpu.async_copy(src_hbm.at[pl.ds(base + g * C, C)], sidx_r[b],
                         sem_i[b])
        pltpu.async_copy(dst_hbm.at[pl.ds(base + g * C, C)], didx_r[b],
                         sem_i[b])

    def wait_idx(g, b):
        pltpu.make_async_copy(src_hbm.at[pl.ds(base + g * C, C)], sidx_r[b],
                              sem_i[b]).wait()
        pltpu.make_async_copy(dst_hbm.at[pl.ds(base + g * C, C)], didx_r[b],
                              sem_i[b]).wait()

    def issue_gather(b):
        pltpu.async_copy(h_hbm.at[sidx_r[b]], grows[b], sem_g[b])
        pltpu.async_copy(as_hbm.at[sidx_r[b]], av_r[b], sem_g[b])
        pltpu.async_copy(ad_hbm.at[didx_r[b]], dv_r[b], sem_g[b])

    def wait_gather(b):
        pltpu.make_async_copy(h_hbm.at[sidx_r[b]], grows[b], sem_g[b]).wait()
        pltpu.make_async_copy(as_hbm.at[sidx_r[b]], av_r[b], sem_g[b]).wait()
        pltpu.make_async_copy(ad_hbm.at[didx_r[b]], dv_r[b], sem_g[b]).wait()

    def scatter_desc(b):
        return pltpu.make_async_copy(grows[b], acc_sh.at[dscat_r[b]],
                                     sem_s[b])

    issue_idx(0, 0)
    issue_idx(1, 1)
    wait_idx(0, 0)
    issue_gather(0)

    def outer(o, carry):
        for b in range(NBUF):
            g = o * NBUF + b
            b1 = (b + 1) % NBUF
            b2 = (b + 2) % NBUF

            @pl.when(g + 2 < NCHUNK)
            def _():
                issue_idx(g + 2, b2)

            @pl.when(g + 1 < NCHUNK)
            def _():
                wait_idx(g + 1, b1)

            @pl.when((g >= 2) & (g + 1 < NCHUNK))
            def _():
                scatter_desc(b1).wait()

            @pl.when(g + 1 < NCHUNK)
            def _():
                issue_gather(b1)

            @pl.when(g < NCHUNK)
            def _():
                wait_gather(b)
                for k in range(C // L):
                    a = av_r[b][pl.ds(k * L, L)]
                    d = dv_r[b][pl.ds(k * L, L)]
                    di = didx_r[b][pl.ds(k * L, L)]
                    y = a + d
                    z = jnp.maximum(y, 0.2 * y)
                    w = jnp.exp(z - mvec)
                    w_r[b][pl.ds(k * L, L)] = w
                    dscat_r[b][pl.ds(k * L, L)] = di
                    plsc.addupdate_scatter(den_v, [di], w)

                def srow(j, carry2, b=b):
                    wj = plsc.load_gather(w_r[b],
                                          [jnp.full((L,), j, jnp.int32)])
                    for k in range(D // L):
                        grows[b][j, pl.ds(k * L, L)] = (
                            grows[b][j, pl.ds(k * L, L)] * wj)
                    return carry2

                lax.fori_loop(0, C, srow, 0)
                scatter_desc(b).start(add=True)
        return carry

    NOUTER = (NCHUNK + NBUF - 1) // NBUF
    lax.fori_loop(0, NOUTER, outer, 0)
    for g in (NCHUNK - 3, NCHUNK - 2, NCHUNK - 1):
        scatter_desc(g % NBUF).wait()
    pltpu.sync_copy(den_v, den_out.at[wid])
    plsc.subcore_barrier()
    pltpu.sync_copy(acc_sh.at[pl.ds(s * RPT, RPT)],
                    acc_out.at[c, pl.ds(s * RPT, RPT)])


def _sc_edges(src, dst, as_, ad_, m, h, zeros_nd, zeros_n):
    N, D = h.shape
    E = src.shape[0]
    C = 80
    NP = ((N + NS * 8 - 1) // (NS * 8)) * NS * 8
    assert E % (NW * C) == 0
    f32 = jnp.float32
    mesh = plsc.VectorSubcoreMesh(core_axis_name="c", subcore_axis_name="s",
                                  num_cores=NC, num_subcores=NS)
    body = functools.partial(_sc_body, N=N, NP=NP, D=D, E=E, C=C)
    fn = pl.kernel(
        body,
        out_type=[
            jax.ShapeDtypeStruct((NC, NP, D), f32),
            jax.ShapeDtypeStruct((NW, N), f32),
        ],
        mesh=mesh,
        compiler_params=pltpu.CompilerParams(needs_layout_passes=False),
        scratch_types=(
            [
                pltpu.VMEM((L,), f32),
                pltpu.VMEM((N,), f32),
            ]
            + [pltpu.VMEM((C,), jnp.int32) for _ in range(NBUF)]
            + [pltpu.VMEM((C,), jnp.int32) for _ in range(NBUF)]
            + [pltpu.VMEM((C,), f32) for _ in range(NBUF)]
            + [pltpu.VMEM((C,), f32) for _ in range(NBUF)]
            + [pltpu.VMEM((C,), f32) for _ in range(NBUF)]
            + [pltpu.VMEM((C,), jnp.int32) for _ in range(NBUF)]
            + [pltpu.VMEM((C, D), f32) for _ in range(NBUF)]
            + [pltpu.VMEM_SHARED((NP, D), f32)]
            + [pltpu.SemaphoreType.DMA for _ in range(3 * NBUF)]
        ),
    )
    return fn(src, dst, as_.reshape(N), ad_.reshape(N), m.reshape(L), h,
              zeros_nd, zeros_n)


def _gat_layer(x, edge_src, edge_dst, W, a_src, a_dst, b, zeros_nd, zeros_n,
               act):
    h, as_, ad_, m, ws = _head(x, W, a_src, a_dst)
    acc, den_parts = _sc_edges(edge_src, edge_dst, as_, ad_, m, h, zeros_nd,
                               zeros_n)
    return _combine(acc, den_parts, h, ws, b, act)


def kernel(x, edge_index, W1, a_src1, a_dst1, b1, W2, a_src2, a_dst2, b2):
    N, D = x.shape
    src = edge_index[0]
    dst = edge_index[1]
    NP = ((N + NS * 8 - 1) // (NS * 8)) * NS * 8
    zeros_nd = jnp.zeros((NP // NS, D), jnp.float32)
    zeros_n = jnp.zeros((N,), jnp.float32)
    h1 = _gat_layer(x, src, dst, W1, a_src1, a_dst1, b1, zeros_nd, zeros_n,
                    "relu")
    return _gat_layer(h1, src, dst, W2, a_src2, a_dst2, b2, zeros_nd,
                      zeros_n, "logsoftmax")


# speedup vs baseline: 50.1678x; 1.0150x over previous
import functools

import jax
import jax.numpy as jnp
from jax import lax
from jax.experimental import pallas as pl
from jax.experimental.pallas import tpu as pltpu
from jax.experimental.pallas import tpu_sc as plsc

NC = 2
NS = 16
L = 16
NW = NC * NS


def _head_body(x_ref, w_ref, asr_ref, adr_ref, h_ref, as_ref, ad_ref,
               m_ref, ws_ref):
    h = jnp.dot(x_ref[...], w_ref[...], preferred_element_type=jnp.float32)
    h_ref[...] = h
    as_ = jnp.sum(h * asr_ref[...], axis=1, keepdims=True)
    ad_ = jnp.sum(h * adr_ref[...], axis=1, keepdims=True)
    as_ref[...] = as_
    ad_ref[...] = ad_
    m = jnp.max(as_) + jnp.max(ad_)
    m = jnp.where(m >= 0.0, m, 0.2 * m)
    m_ref[...] = jnp.full((1, L), m, jnp.float32)
    y = as_ + ad_
    z = jnp.maximum(y, 0.2 * y)
    ws_ref[...] = jnp.exp(z - m)


def _head(x, W, a_src, a_dst):
    N, D = x.shape
    f32 = jnp.float32
    return pl.pallas_call(
        _head_body,
        out_shape=[
            jax.ShapeDtypeStruct((N, D), f32),
            jax.ShapeDtypeStruct((N, 1), f32),
            jax.ShapeDtypeStruct((N, 1), f32),
            jax.ShapeDtypeStruct((1, L), f32),
            jax.ShapeDtypeStruct((N, 1), f32),
        ],
    )(x, W, a_src.reshape(1, D), a_dst.reshape(1, D))


def _combine_body(acc_ref, dent_ref, h_ref, ws_ref, b_ref, o_ref, *, act, D):
    ws = ws_ref[...]
    N = ws_ref.shape[0]
    num = acc_ref[0, :N, :] + acc_ref[1, :N, :] + ws * h_ref[...]
    den = jnp.sum(dent_ref[...], axis=1, keepdims=True) + ws
    out = num / (den + 1e-16) + b_ref[...]
    if act == "relu":
        out = jnp.maximum(out, 0.0)
    else:
        mx = jnp.max(out, axis=1, keepdims=True)
        sh = out - mx
        out = sh - jnp.log(jnp.sum(jnp.exp(sh), axis=1, keepdims=True))
    o_ref[...] = out


def _combine(acc, den_parts, h, ws, b, act):
    N, D = h.shape
    den_t = den_parts.T
    return pl.pallas_call(
        functools.partial(_combine_body, act=act, D=D),
        out_shape=jax.ShapeDtypeStruct((N, D), jnp.float32),
    )(acc, den_t, h, ws, b.reshape(1, D))


NBUF = 3


def _sc_body(src_hbm, dst_hbm, as_hbm, ad_hbm, m_hbm, h_hbm, z_hbm, zn_hbm,
             acc_out, den_out, m_v, den_v, *rings, N, NP, D, E, C):
    RPT = NP // NS
    EPW = E // NW
    NCHUNK = EPW // C
    sidx_r = rings[0:NBUF]
    didx_r = rings[NBUF:2 * NBUF]
    av_r = rings[2 * NBUF:3 * NBUF]
    dv_r = rings[3 * NBUF:4 * NBUF]
    w_r = rings[4 * NBUF:5 * NBUF]
    dscat_r = rings[5 * NBUF:6 * NBUF]
    grows = rings[6 * NBUF:7 * NBUF]
    acc_sh = rings[7 * NBUF]
    sem_i = rings[7 * NBUF + 1:8 * NBUF + 1]
    sem_g = rings[8 * NBUF + 1:9 * NBUF + 1]
    sem_s = rings[9 * NBUF + 1:10 * NBUF + 1]

    c = lax.axis_index("c")
    s = lax.axis_index("s")
    wid = s * NC + c
    base = wid * EPW

    pltpu.sync_copy(m_hbm, m_v)
    pltpu.sync_copy(zn_hbm, den_v)
    pltpu.sync_copy(z_hbm, acc_sh.at[pl.ds(s * RPT, RPT)])
    plsc.subcore_barrier()
    mvec = m_v[...]

    def issue_idx(g, b):
        pltpu.async_copy(src_hbm.at[pl.ds(base + g * C, C)], sidx_r[b],
                         sem_i[b])
        pltpu.async_copy(dst_hbm.at[pl.ds(base + g * C, C)], didx_r[b],
                         sem_i[b])

    def wait_idx(g, b):
        pltpu.make_async_copy(src_hbm.at[pl.ds(base + g * C, C)], sidx_r[b],
                              sem_i[b]).wait()
        pltpu.make_async_copy(dst_hbm.at[pl.ds(base + g * C, C)], didx_r[b],
                              sem_i[b]).wait()

    def issue_gather(b):
        pltpu.async_copy(h_hbm.at[sidx_r[b]], grows[b], sem_g[b])
        pltpu.async_copy(as_hbm.at[sidx_r[b]], av_r[b], sem_g[b])
        pltpu.async_copy(ad_hbm.at[didx_r[b]], dv_r[b], sem_g[b])

    def wait_gather(b):
        pltpu.make_async_copy(h_hbm.at[sidx_r[b]], grows[b], sem_g[b]).wait()
        pltpu.make_async_copy(as_hbm.at[sidx_r[b]], av_r[b], sem_g[b]).wait()
        pltpu.make_async_copy(ad_hbm.at[didx_r[b]], dv_r[b], sem_g[b]).wait()

    def scatter_desc(b):
        return pltpu.make_async_copy(grows[b], acc_sh.at[dscat_r[b]],
                                     sem_s[b])

    issue_idx(0, 0)
    issue_idx(1, 1)
    wait_idx(0, 0)
    issue_gather(0)

    def outer(o, carry):
        for b in range(NBUF):
            g = o * NBUF + b
            b1 = (b + 1) % NBUF
            b2 = (b + 2) % NBUF

            @pl.when(g + 2 < NCHUNK)
            def _():
                issue_idx(g + 2, b2)

            @pl.when(g + 1 < NCHUNK)
            def _():
                wait_idx(g + 1, b1)

            @pl.when((g >= 2) & (g + 1 < NCHUNK))
            def _():
                scatter_desc(b1).wait()

            @pl.when(g + 1 < NCHUNK)
            def _():
                issue_gather(b1)

            @pl.when(g < NCHUNK)
            def _():
                wait_gather(b)
                for k in range(C // L):
                    a = av_r[b][pl.ds(k * L, L)]
                    d = dv_r[b][pl.ds(k * L, L)]
                    di = didx_r[b][pl.ds(k * L, L)]
                    y = a + d
                    z = jnp.maximum(y, 0.2 * y)
                    w = jnp.exp(z - mvec)
                    w_r[b][pl.ds(k * L, L)] = w
                    dscat_r[b][pl.ds(k * L, L)] = di
                    plsc.addupdate_scatter(den_v, [di], w)

                def srow(j, carry2, b=b):
                    wj = plsc.load_gather(w_r[b],
                                          [jnp.full((L,), j, jnp.int32)])
                    for k in range(D // L):
                        grows[b][j, pl.ds(k * L, L)] = (
                            grows[b][j, pl.ds(k * L, L)] * wj)
                    return carry2

                lax.fori_loop(0, C, srow, 0, unroll=4)
                scatter_desc(b).start(add=True)
        return carry

    NOUTER = (NCHUNK + NBUF - 1) // NBUF
    lax.fori_loop(0, NOUTER, outer, 0)
    for g in (NCHUNK - 3, NCHUNK - 2, NCHUNK - 1):
        scatter_desc(g % NBUF).wait()
    pltpu.sync_copy(den_v, den_out.at[wid])
    plsc.subcore_barrier()
    pltpu.sync_copy(acc_sh.at[pl.ds(s * RPT, RPT)],
                    acc_out.at[c, pl.ds(s * RPT, RPT)])


def _sc_edges(edge_index, as_, ad_, m, h, zeros_nd, zeros_n):
    N, D = h.shape
    E = edge_index.shape[1]
    C = 80
    NP = ((N + NS * 8 - 1) // (NS * 8)) * NS * 8
    assert E % (NW * C) == 0
    f32 = jnp.float32
    mesh = plsc.VectorSubcoreMesh(core_axis_name="c", subcore_axis_name="s",
                                  num_cores=NC, num_subcores=NS)
    body = functools.partial(_sc_body, N=N, NP=NP, D=D, E=E, C=C)
    fn = pl.kernel(
        body,
        out_type=[
            jax.ShapeDtypeStruct((NC, NP, D), f32),
            jax.ShapeDtypeStruct((NW, N), f32),
        ],
        mesh=mesh,
        compiler_params=pltpu.CompilerParams(needs_layout_passes=False),
        scratch_types=(
            [
                pltpu.VMEM((L,), f32),
                pltpu.VMEM((N,), f32),
            ]
            + [pltpu.VMEM((C,), jnp.int32) for _ in range(NBUF)]
            + [pltpu.VMEM((C,), jnp.int32) for _ in range(NBUF)]
            + [pltpu.VMEM((C,), f32) for _ in range(NBUF)]
            + [pltpu.VMEM((C,), f32) for _ in range(NBUF)]
            + [pltpu.VMEM((C,), f32) for _ in range(NBUF)]
            + [pltpu.VMEM((C,), jnp.int32) for _ in range(NBUF)]
            + [pltpu.VMEM((C, D), f32) for _ in range(NBUF)]
            + [pltpu.VMEM_SHARED((NP, D), f32)]
            + [pltpu.SemaphoreType.DMA for _ in range(3 * NBUF)]
        ),
    )
    return fn(edge_index[0], edge_index[1], as_.reshape(N), ad_.reshape(N),
              m.reshape(L), h, zeros_nd, zeros_n)


def _gat_layer(x, edge_index, W, a_src, a_dst, b, zeros_nd, zeros_n, act):
    h, as_, ad_, m, ws = _head(x, W, a_src, a_dst)
    acc, den_parts = _sc_edges(edge_index, as_, ad_, m, h, zeros_nd, zeros_n)
    return _combine(acc, den_parts, h, ws, b, act)


def kernel(x, edge_index, W1, a_src1, a_dst1, b1, W2, a_src2, a_dst2, b2):
    N, D = x.shape
    NP = ((N + NS * 8 - 1) // (NS * 8)) * NS * 8
    zeros_nd = jnp.zeros((NP // NS, D), jnp.float32)
    zeros_n = jnp.zeros((N,), jnp.float32)
    h1 = _gat_layer(x, edge_index, W1, a_src1, a_dst1, b1, zeros_nd, zeros_n,
                    "relu")
    return _gat_layer(h1, edge_index, W2, a_src2, a_dst2, b2, zeros_nd,
                      zeros_n, "logsoftmax")


# speedup vs baseline: 50.6813x; 1.0102x over previous
import functools

import jax
import jax.numpy as jnp
from jax import lax
from jax.experimental import pallas as pl
from jax.experimental.pallas import tpu as pltpu
from jax.experimental.pallas import tpu_sc as plsc

NC = 2
NS = 16
L = 16
NW = NC * NS


def _head_body(x_ref, w_ref, asr_ref, adr_ref, h_ref, as_ref, ad_ref,
               m_ref, ws_ref):
    h = jnp.dot(x_ref[...], w_ref[...], preferred_element_type=jnp.float32)
    h_ref[...] = h
    as_ = jnp.sum(h * asr_ref[...], axis=1, keepdims=True)
    ad_ = jnp.sum(h * adr_ref[...], axis=1, keepdims=True)
    as_ref[...] = as_
    ad_ref[...] = ad_
    m = jnp.max(as_) + jnp.max(ad_)
    m = jnp.where(m >= 0.0, m, 0.2 * m)
    m_ref[...] = jnp.full((1, L), m, jnp.float32)
    y = as_ + ad_
    z = jnp.maximum(y, 0.2 * y)
    ws_ref[...] = jnp.exp(z - m)


def _head(x, W, a_src, a_dst):
    N, D = x.shape
    f32 = jnp.float32
    return pl.pallas_call(
        _head_body,
        out_shape=[
            jax.ShapeDtypeStruct((N, D), f32),
            jax.ShapeDtypeStruct((N, 1), f32),
            jax.ShapeDtypeStruct((N, 1), f32),
            jax.ShapeDtypeStruct((1, L), f32),
            jax.ShapeDtypeStruct((N, 1), f32),
        ],
    )(x, W, a_src.reshape(1, D), a_dst.reshape(1, D))


def _combine_body(acc_ref, dent_ref, h_ref, ws_ref, b_ref, o_ref, *, act, D):
    ws = ws_ref[...]
    N = ws_ref.shape[0]
    num = acc_ref[0, :N, :] + acc_ref[1, :N, :] + ws * h_ref[...]
    den = jnp.sum(dent_ref[...], axis=1, keepdims=True) + ws
    out = num / (den + 1e-16) + b_ref[...]
    if act == "relu":
        out = jnp.maximum(out, 0.0)
    else:
        mx = jnp.max(out, axis=1, keepdims=True)
        sh = out - mx
        out = sh - jnp.log(jnp.sum(jnp.exp(sh), axis=1, keepdims=True))
    o_ref[...] = out


def _combine(acc, den_parts, h, ws, b, act):
    N, D = h.shape
    den_t = den_parts.T
    return pl.pallas_call(
        functools.partial(_combine_body, act=act, D=D),
        out_shape=jax.ShapeDtypeStruct((N, D), jnp.float32),
    )(acc, den_t, h, ws, b.reshape(1, D))


def _mid_body(acc_ref, dent_ref, h_ref, ws_ref, b_ref, w2_ref, asr2_ref,
              adr2_ref, h2_ref, as2_ref, ad2_ref, m2_ref, ws2_ref, *, D):
    ws = ws_ref[...]
    N = ws_ref.shape[0]
    num = acc_ref[0, :N, :] + acc_ref[1, :N, :] + ws * h_ref[...]
    den = jnp.sum(dent_ref[...], axis=1, keepdims=True) + ws
    o1 = num / (den + 1e-16) + b_ref[...]
    o1 = jnp.maximum(o1, 0.0)
    h2 = jnp.dot(o1, w2_ref[...], preferred_element_type=jnp.float32)
    h2_ref[...] = h2
    as2 = jnp.sum(h2 * asr2_ref[...], axis=1, keepdims=True)
    ad2 = jnp.sum(h2 * adr2_ref[...], axis=1, keepdims=True)
    as2_ref[...] = as2
    ad2_ref[...] = ad2
    m2 = jnp.max(as2) + jnp.max(ad2)
    m2 = jnp.where(m2 >= 0.0, m2, 0.2 * m2)
    m2_ref[...] = jnp.full((1, L), m2, jnp.float32)
    y2 = as2 + ad2
    z2 = jnp.maximum(y2, 0.2 * y2)
    ws2_ref[...] = jnp.exp(z2 - m2)


def _mid(acc, den_parts, h, ws, b, W2, a_src2, a_dst2):
    N, D = h.shape
    den_t = den_parts.T
    f32 = jnp.float32
    return pl.pallas_call(
        functools.partial(_mid_body, D=D),
        out_shape=[
            jax.ShapeDtypeStruct((N, D), f32),
            jax.ShapeDtypeStruct((N, 1), f32),
            jax.ShapeDtypeStruct((N, 1), f32),
            jax.ShapeDtypeStruct((1, L), f32),
            jax.ShapeDtypeStruct((N, 1), f32),
        ],
    )(acc, den_t, h, ws, b.reshape(1, D), W2, a_src2.reshape(1, D),
      a_dst2.reshape(1, D))


NBUF = 3


def _sc_body(src_hbm, dst_hbm, as_hbm, ad_hbm, m_hbm, h_hbm, z_hbm, zn_hbm,
             acc_out, den_out, m_v, den_v, *rings, N, NP, D, E, C):
    RPT = NP // NS
    EPW = E // NW
    NCHUNK = EPW // C
    sidx_r = rings[0:NBUF]
    didx_r = rings[NBUF:2 * NBUF]
    av_r = rings[2 * NBUF:3 * NBUF]
    dv_r = rings[3 * NBUF:4 * NBUF]
    w_r = rings[4 * NBUF:5 * NBUF]
    dscat_r = rings[5 * NBUF:6 * NBUF]
    grows = rings[6 * NBUF:7 * NBUF]
    acc_sh = rings[7 * NBUF]
    sem_i = rings[7 * NBUF + 1:8 * NBUF + 1]
    sem_g = rings[8 * NBUF + 1:9 * NBUF + 1]
    sem_s = rings[9 * NBUF + 1:10 * NBUF + 1]

    c = lax.axis_index("c")
    s = lax.axis_index("s")
    wid = s * NC + c
    base = wid * EPW

    pltpu.sync_copy(m_hbm, m_v)
    pltpu.sync_copy(zn_hbm, den_v)
    pltpu.sync_copy(z_hbm, acc_sh.at[pl.ds(s * RPT, RPT)])
    plsc.subcore_barrier()
    mvec = m_v[...]

    def issue_idx(g, b):
        pltpu.async_copy(src_hbm.at[pl.ds(base + g * C, C)], sidx_r[b],
                         sem_i[b])
        pltpu.async_copy(dst_hbm.at[pl.ds(base + g * C, C)], didx_r[b],
                         sem_i[b])

    def wait_idx(g, b):
        pltpu.make_async_copy(src_hbm.at[pl.ds(base + g * C, C)], sidx_r[b],
                              sem_i[b]).wait()
        pltpu.make_async_copy(dst_hbm.at[pl.ds(base + g * C, C)], didx_r[b],
                              sem_i[b]).wait()

    NSPL = 2
    CS = C // NSPL

    def issue_gather(b):
        for q in range(NSPL):
            pltpu.async_copy(h_hbm.at[sidx_r[b].at[pl.ds(q * CS, CS)]],
                             grows[b].at[pl.ds(q * CS, CS)], sem_g[b])
        pltpu.async_copy(as_hbm.at[sidx_r[b]], av_r[b], sem_g[b])
        pltpu.async_copy(ad_hbm.at[didx_r[b]], dv_r[b], sem_g[b])

    def wait_gather(b):
        for q in range(NSPL):
            pltpu.make_async_copy(h_hbm.at[sidx_r[b].at[pl.ds(q * CS, CS)]],
                                  grows[b].at[pl.ds(q * CS, CS)],
                                  sem_g[b]).wait()
        pltpu.make_async_copy(as_hbm.at[sidx_r[b]], av_r[b], sem_g[b]).wait()
        pltpu.make_async_copy(ad_hbm.at[didx_r[b]], dv_r[b], sem_g[b]).wait()

    def scatter_desc(b):
        return pltpu.make_async_copy(grows[b], acc_sh.at[dscat_r[b]],
                                     sem_s[b])

    issue_idx(0, 0)
    issue_idx(1, 1)
    wait_idx(0, 0)
    issue_gather(0)

    def outer(o, carry):
        for b in range(NBUF):
            g = o * NBUF + b
            b1 = (b + 1) % NBUF
            b2 = (b + 2) % NBUF

            @pl.when(g + 2 < NCHUNK)
            def _():
                issue_idx(g + 2, b2)

            @pl.when(g + 1 < NCHUNK)
            def _():
                wait_idx(g + 1, b1)

            @pl.when((g >= 2) & (g + 1 < NCHUNK))
            def _():
                scatter_desc(b1).wait()

            @pl.when(g + 1 < NCHUNK)
            def _():
                issue_gather(b1)

            @pl.when(g < NCHUNK)
            def _():
                wait_gather(b)
                for k in range(C // L):
                    a = av_r[b][pl.ds(k * L, L)]
                    d = dv_r[b][pl.ds(k * L, L)]
                    di = didx_r[b][pl.ds(k * L, L)]
                    y = a + d
                    z = jnp.maximum(y, 0.2 * y)
                    w = jnp.exp(z - mvec)
                    w_r[b][pl.ds(k * L, L)] = w
                    dscat_r[b][pl.ds(k * L, L)] = di
                    plsc.addupdate_scatter(den_v, [di], w)

                def srow(j, carry2, b=b):
                    wj = plsc.load_gather(w_r[b],
                                          [jnp.full((L,), j, jnp.int32)])
                    for k in range(D // L):
                        grows[b][j, pl.ds(k * L, L)] = (
                            grows[b][j, pl.ds(k * L, L)] * wj)
                    return carry2

                lax.fori_loop(0, C, srow, 0, unroll=4)
                scatter_desc(b).start(add=True)
        return carry

    NOUTER = (NCHUNK + NBUF - 1) // NBUF
    lax.fori_loop(0, NOUTER, outer, 0)
    for g in (NCHUNK - 3, NCHUNK - 2, NCHUNK - 1):
        scatter_desc(g % NBUF).wait()
    pltpu.sync_copy(den_v, den_out.at[wid])
    plsc.subcore_barrier()
    pltpu.sync_copy(acc_sh.at[pl.ds(s * RPT, RPT)],
                    acc_out.at[c, pl.ds(s * RPT, RPT)])


def _sc_edges(edge_index, as_, ad_, m, h, zeros_nd, zeros_n):
    N, D = h.shape
    E = edge_index.shape[1]
    C = 80
    NP = ((N + NS * 8 - 1) // (NS * 8)) * NS * 8
    assert E % (NW * C) == 0
    f32 = jnp.float32
    mesh = plsc.VectorSubcoreMesh(core_axis_name="c", subcore_axis_name="s",
                                  num_cores=NC, num_subcores=NS)
    body = functools.partial(_sc_body, N=N, NP=NP, D=D, E=E, C=C)
    fn = pl.kernel(
        body,
        out_type=[
            jax.ShapeDtypeStruct((NC, NP, D), f32),
            jax.ShapeDtypeStruct((NW, N), f32),
        ],
        mesh=mesh,
        compiler_params=pltpu.CompilerParams(needs_layout_passes=False),
        scratch_types=(
            [
                pltpu.VMEM((L,), f32),
                pltpu.VMEM((N,), f32),
            ]
            + [pltpu.VMEM((C,), jnp.int32) for _ in range(NBUF)]
            + [pltpu.VMEM((C,), jnp.int32) for _ in range(NBUF)]
            + [pltpu.VMEM((C,), f32) for _ in range(NBUF)]
            + [pltpu.VMEM((C,), f32) for _ in range(NBUF)]
            + [pltpu.VMEM((C,), f32) for _ in range(NBUF)]
            + [pltpu.VMEM((C,), jnp.int32) for _ in range(NBUF)]
            + [pltpu.VMEM((C, D), f32) for _ in range(NBUF)]
            + [pltpu.VMEM_SHARED((NP, D), f32)]
            + [pltpu.SemaphoreType.DMA for _ in range(3 * NBUF)]
        ),
    )
    return fn(edge_index[0], edge_index[1], as_.reshape(N), ad_.reshape(N),
              m.reshape(L), h, zeros_nd, zeros_n)


def kernel(x, edge_index, W1, a_src1, a_dst1, b1, W2, a_src2, a_dst2, b2):
    N, D = x.shape
    NP = ((N + NS * 8 - 1) // (NS * 8)) * NS * 8
    zeros_nd = jnp.zeros((NP // NS, D), jnp.float32)
    zeros_n = jnp.zeros((N,), jnp.float32)
    h1, as1, ad1, m1, ws1 = _head(x, W1, a_src1, a_dst1)
    acc1, den1 = _sc_edges(edge_index, as1, ad1, m1, h1, zeros_nd, zeros_n)
    h2, as2, ad2, m2, ws2 = _mid(acc1, den1, h1, ws1, b1, W2, a_src2,
                                 a_dst2)
    acc2, den2 = _sc_edges(edge_index, as2, ad2, m2, h2, zeros_nd, zeros_n)
    return _combine(acc2, den2, h2, ws2, b2, "logsoftmax")


# speedup vs baseline: 50.8680x; 1.0037x over previous
import functools

import jax
import jax.numpy as jnp
from jax import lax
from jax.experimental import pallas as pl
from jax.experimental.pallas import tpu as pltpu
from jax.experimental.pallas import tpu_sc as plsc

NC = 2
NS = 16
L = 16
NW = NC * NS


def _head_body(x_ref, w_ref, asr_ref, adr_ref, h_ref, as_ref, ad_ref,
               m_ref, ws_ref):
    h = jnp.dot(x_ref[...], w_ref[...], preferred_element_type=jnp.float32)
    h_ref[...] = h
    as_ = jnp.sum(h * asr_ref[...], axis=1, keepdims=True)
    ad_ = jnp.sum(h * adr_ref[...], axis=1, keepdims=True)
    as_ref[...] = as_
    ad_ref[...] = ad_
    m = jnp.max(as_) + jnp.max(ad_)
    m = jnp.where(m >= 0.0, m, 0.2 * m)
    m_ref[...] = jnp.full((1, L), m, jnp.float32)
    y = as_ + ad_
    z = jnp.maximum(y, 0.2 * y)
    ws_ref[...] = jnp.exp(z - m)


def _head(x, W, a_src, a_dst):
    N, D = x.shape
    f32 = jnp.float32
    return pl.pallas_call(
        _head_body,
        out_shape=[
            jax.ShapeDtypeStruct((N, D), f32),
            jax.ShapeDtypeStruct((N, 1), f32),
            jax.ShapeDtypeStruct((N, 1), f32),
            jax.ShapeDtypeStruct((1, L), f32),
            jax.ShapeDtypeStruct((N, 1), f32),
        ],
    )(x, W, a_src.reshape(1, D), a_dst.reshape(1, D))


def _combine_body(acc_ref, dent_ref, h_ref, ws_ref, b_ref, o_ref, *, act, D):
    ws = ws_ref[...]
    N = ws_ref.shape[0]
    num = acc_ref[0, :N, :] + acc_ref[1, :N, :] + ws * h_ref[...]
    den = jnp.sum(dent_ref[...], axis=1, keepdims=True) + ws
    out = num / (den + 1e-16) + b_ref[...]
    if act == "relu":
        out = jnp.maximum(out, 0.0)
    else:
        mx = jnp.max(out, axis=1, keepdims=True)
        sh = out - mx
        out = sh - jnp.log(jnp.sum(jnp.exp(sh), axis=1, keepdims=True))
    o_ref[...] = out


def _combine(acc, den_parts, h, ws, b, act):
    N, D = h.shape
    den_t = den_parts.T
    return pl.pallas_call(
        functools.partial(_combine_body, act=act, D=D),
        out_shape=jax.ShapeDtypeStruct((N, D), jnp.float32),
    )(acc, den_t, h, ws, b.reshape(1, D))


def _mid_body(acc_ref, dent_ref, h_ref, ws_ref, b_ref, w2_ref, asr2_ref,
              adr2_ref, h2_ref, as2_ref, ad2_ref, m2_ref, ws2_ref, *, D):
    ws = ws_ref[...]
    N = ws_ref.shape[0]
    num = acc_ref[0, :N, :] + acc_ref[1, :N, :] + ws * h_ref[...]
    den = jnp.sum(dent_ref[...], axis=1, keepdims=True) + ws
    o1 = num / (den + 1e-16) + b_ref[...]
    o1 = jnp.maximum(o1, 0.0)
    h2 = jnp.dot(o1, w2_ref[...], preferred_element_type=jnp.float32)
    h2_ref[...] = h2
    as2 = jnp.sum(h2 * asr2_ref[...], axis=1, keepdims=True)
    ad2 = jnp.sum(h2 * adr2_ref[...], axis=1, keepdims=True)
    as2_ref[...] = as2
    ad2_ref[...] = ad2
    m2 = jnp.max(as2) + jnp.max(ad2)
    m2 = jnp.where(m2 >= 0.0, m2, 0.2 * m2)
    m2_ref[...] = jnp.full((1, L), m2, jnp.float32)
    y2 = as2 + ad2
    z2 = jnp.maximum(y2, 0.2 * y2)
    ws2_ref[...] = jnp.exp(z2 - m2)


def _mid(acc, den_parts, h, ws, b, W2, a_src2, a_dst2):
    N, D = h.shape
    den_t = den_parts.T
    f32 = jnp.float32
    return pl.pallas_call(
        functools.partial(_mid_body, D=D),
        out_shape=[
            jax.ShapeDtypeStruct((N, D), f32),
            jax.ShapeDtypeStruct((N, 1), f32),
            jax.ShapeDtypeStruct((N, 1), f32),
            jax.ShapeDtypeStruct((1, L), f32),
            jax.ShapeDtypeStruct((N, 1), f32),
        ],
    )(acc, den_t, h, ws, b.reshape(1, D), W2, a_src2.reshape(1, D),
      a_dst2.reshape(1, D))


NBUF = 3


def _sc_body(src_hbm, dst_hbm, as_hbm, ad_hbm, m_hbm, h_hbm, z_hbm, zn_hbm,
             acc_out, den_out, m_v, den_v, *rings, N, NP, D, E, C):
    RPT = NP // NS
    EPW = E // NW
    NCHUNK = EPW // C
    sidx_r = rings[0:NBUF]
    didx_r = rings[NBUF:2 * NBUF]
    av_r = rings[2 * NBUF:3 * NBUF]
    dv_r = rings[3 * NBUF:4 * NBUF]
    w_r = rings[4 * NBUF:5 * NBUF]
    dscat_r = rings[5 * NBUF:6 * NBUF]
    grows = rings[6 * NBUF:7 * NBUF]
    acc_sh = rings[7 * NBUF]
    sem_i = rings[7 * NBUF + 1:8 * NBUF + 1]
    sem_g = rings[8 * NBUF + 1:9 * NBUF + 1]
    sem_s = rings[9 * NBUF + 1:10 * NBUF + 1]

    c = lax.axis_index("c")
    s = lax.axis_index("s")
    wid = s * NC + c
    base = wid * EPW

    pltpu.sync_copy(m_hbm, m_v)
    pltpu.sync_copy(zn_hbm, den_v)
    pltpu.sync_copy(z_hbm, acc_sh.at[pl.ds(s * RPT, RPT)])
    plsc.subcore_barrier()
    mvec = m_v[...]

    def issue_idx(g, b):
        pltpu.async_copy(src_hbm.at[pl.ds(base + g * C, C)], sidx_r[b],
                         sem_i[b])
        pltpu.async_copy(dst_hbm.at[pl.ds(base + g * C, C)], didx_r[b],
                         sem_i[b])

    def wait_idx(g, b):
        pltpu.make_async_copy(src_hbm.at[pl.ds(base + g * C, C)], sidx_r[b],
                              sem_i[b]).wait()
        pltpu.make_async_copy(dst_hbm.at[pl.ds(base + g * C, C)], didx_r[b],
                              sem_i[b]).wait()

    NSPL = 1
    CS = C // NSPL

    def issue_gather(b):
        for q in range(NSPL):
            pltpu.async_copy(h_hbm.at[sidx_r[b].at[pl.ds(q * CS, CS)]],
                             grows[b].at[pl.ds(q * CS, CS)], sem_g[b])
        pltpu.async_copy(as_hbm.at[sidx_r[b]], av_r[b], sem_g[b])
        pltpu.async_copy(ad_hbm.at[didx_r[b]], dv_r[b], sem_g[b])

    def wait_gather(b):
        for q in range(NSPL):
            pltpu.make_async_copy(h_hbm.at[sidx_r[b].at[pl.ds(q * CS, CS)]],
                                  grows[b].at[pl.ds(q * CS, CS)],
                                  sem_g[b]).wait()
        pltpu.make_async_copy(as_hbm.at[sidx_r[b]], av_r[b], sem_g[b]).wait()
        pltpu.make_async_copy(ad_hbm.at[didx_r[b]], dv_r[b], sem_g[b]).wait()

    def scatter_desc(b):
        return pltpu.make_async_copy(grows[b], acc_sh.at[dscat_r[b]],
                                     sem_s[b])

    issue_idx(0, 0)
    issue_idx(1, 1)
    wait_idx(0, 0)
    issue_gather(0)

    def outer(o, carry):
        for b in range(NBUF):
            g = o * NBUF + b
            b1 = (b + 1) % NBUF
            b2 = (b + 2) % NBUF

            @pl.when(g + 2 < NCHUNK)
            def _():
                issue_idx(g + 2, b2)

            @pl.when(g + 1 < NCHUNK)
            def _():
                wait_idx(g + 1, b1)

            @pl.when((g >= 2) & (g + 1 < NCHUNK))
            def _():
                scatter_desc(b1).wait()

            @pl.when(g + 1 < NCHUNK)
            def _():
                issue_gather(b1)

            @pl.when(g < NCHUNK)
            def _():
                wait_gather(b)
                for k in range(C // L):
                    a = av_r[b][pl.ds(k * L, L)]
                    d = dv_r[b][pl.ds(k * L, L)]
                    di = didx_r[b][pl.ds(k * L, L)]
                    y = a + d
                    z = jnp.maximum(y, 0.2 * y)
                    w = jnp.exp(z - mvec)
                    w_r[b][pl.ds(k * L, L)] = w
                    dscat_r[b][pl.ds(k * L, L)] = di
                    plsc.addupdate_scatter(den_v, [di], w)

                def srow(j, carry2, b=b):
                    wj = plsc.load_gather(w_r[b],
                                          [jnp.full((L,), j, jnp.int32)])
                    for k in range(D // L):
                        grows[b][j, pl.ds(k * L, L)] = (
                            grows[b][j, pl.ds(k * L, L)] * wj)
                    return carry2

                lax.fori_loop(0, C, srow, 0, unroll=4)
                scatter_desc(b).start(add=True)
        return carry

    NOUTER = (NCHUNK + NBUF - 1) // NBUF
    lax.fori_loop(0, NOUTER, outer, 0)
    for g in (NCHUNK - 3, NCHUNK - 2, NCHUNK - 1):
        scatter_desc(g % NBUF).wait()
    pltpu.sync_copy(den_v, den_out.at[wid])
    plsc.subcore_barrier()
    pltpu.sync_copy(acc_sh.at[pl.ds(s * RPT, RPT)],
                    acc_out.at[c, pl.ds(s * RPT, RPT)])


def _sc_edges(edge_index, as_, ad_, m, h, zeros_nd, zeros_n):
    N, D = h.shape
    E = edge_index.shape[1]
    C = 80
    NP = ((N + NS * 8 - 1) // (NS * 8)) * NS * 8
    assert E % (NW * C) == 0
    f32 = jnp.float32
    mesh = plsc.VectorSubcoreMesh(core_axis_name="c", subcore_axis_name="s",
                                  num_cores=NC, num_subcores=NS)
    body = functools.partial(_sc_body, N=N, NP=NP, D=D, E=E, C=C)
    fn = pl.kernel(
        body,
        out_type=[
            jax.ShapeDtypeStruct((NC, NP, D), f32),
            jax.ShapeDtypeStruct((NW, N), f32),
        ],
        mesh=mesh,
        compiler_params=pltpu.CompilerParams(needs_layout_passes=False),
        scratch_types=(
            [
                pltpu.VMEM((L,), f32),
                pltpu.VMEM((N,), f32),
            ]
            + [pltpu.VMEM((C,), jnp.int32) for _ in range(NBUF)]
            + [pltpu.VMEM((C,), jnp.int32) for _ in range(NBUF)]
            + [pltpu.VMEM((C,), f32) for _ in range(NBUF)]
            + [pltpu.VMEM((C,), f32) for _ in range(NBUF)]
            + [pltpu.VMEM((C,), f32) for _ in range(NBUF)]
            + [pltpu.VMEM((C,), jnp.int32) for _ in range(NBUF)]
            + [pltpu.VMEM((C, D), f32) for _ in range(NBUF)]
            + [pltpu.VMEM_SHARED((NP, D), f32)]
            + [pltpu.SemaphoreType.DMA for _ in range(3 * NBUF)]
        ),
    )
    return fn(edge_index[0], edge_index[1], as_.reshape(N), ad_.reshape(N),
              m.reshape(L), h, zeros_nd, zeros_n)


def kernel(x, edge_index, W1, a_src1, a_dst1, b1, W2, a_src2, a_dst2, b2):
    N, D = x.shape
    NP = ((N + NS * 8 - 1) // (NS * 8)) * NS * 8
    zeros_nd = jnp.zeros((NP // NS, D), jnp.float32)
    zeros_n = jnp.zeros((N,), jnp.float32)
    h1, as1, ad1, m1, ws1 = _head(x, W1, a_src1, a_dst1)
    acc1, den1 = _sc_edges(edge_index, as1, ad1, m1, h1, zeros_nd, zeros_n)
    h2, as2, ad2, m2, ws2 = _mid(acc1, den1, h1, ws1, b1, W2, a_src2,
                                 a_dst2)
    acc2, den2 = _sc_edges(edge_index, as2, ad2, m2, h2, zeros_nd, zeros_n)
    return _combine(acc2, den2, h2, ws2, b2, "logsoftmax")
